# baseline (speedup 1.0000x reference)
import functools

import jax
import jax.numpy as jnp
from jax import lax
from jax.experimental import pallas as pl
from jax.experimental.pallas import tpu as pltpu
from jax.experimental.pallas import tpu_sc as plsc

B = 4096
F = 26
V = 1000000
D = 16
NUM = 13
H1 = 256
H2 = 128
EPS = 1e-5

NC = 2
NS = 16
NW = NC * NS
R = B * F
RPW = R // NW
CH = 128
NCH = RPW // CH

_mesh = plsc.VectorSubcoreMesh(core_axis_name="c", subcore_axis_name="s")


@functools.partial(
    pl.kernel,
    mesh=_mesh,
    out_type=jax.ShapeDtypeStruct((R, D), jnp.float32),
    scratch_types=[
        pltpu.VMEM((NCH, CH), jnp.int32),
        pltpu.VMEM((RPW, D), jnp.float32),
        pltpu.SemaphoreType.DMA,
    ],
    compiler_params=pltpu.CompilerParams(use_tc_tiling_on_sc=False),
)
def _sc_gather(table_hbm, idx_hbm, out_hbm, idx_v, rows_v, sem):
    wid = lax.axis_index("s") * NC + lax.axis_index("c")
    pltpu.sync_copy(idx_hbm.at[wid], idx_v)
    copies = []
    for j in range(NCH):
        cp = pltpu.make_async_copy(
            table_hbm.at[idx_v.at[j]],
            rows_v.at[pl.ds(j * CH, CH)],
            sem,
        )
        cp.start()
        copies.append(cp)
    for cp in copies:
        cp.wait()
    pltpu.sync_copy(rows_v, out_hbm.at[pl.ds(wid * RPW, RPW)])


def _mlp_body(xc_ref, xn_ref,
              bg_ref, bb_ref, bm_ref, bv_ref,
              w0c_ref, w0n_ref, b0_ref, g0_ref, be0_ref, m0_ref, v0_ref,
              w1_ref, b1_ref, g1_ref, be1_ref, m1_ref, v1_ref,
              w2_ref, b2_ref, out_ref):
    xn = xn_ref[...]
    xnb = (xn - bm_ref[...]) * lax.rsqrt(bv_ref[...] + EPS) * bg_ref[...] + bb_ref[...]
    h = jnp.dot(xc_ref[...], w0c_ref[...], preferred_element_type=jnp.float32)
    h = h + jnp.dot(xnb, w0n_ref[...], preferred_element_type=jnp.float32)
    h = jnp.maximum(h + b0_ref[...], 0.0)
    h = (h - m0_ref[...]) * lax.rsqrt(v0_ref[...] + EPS) * g0_ref[...] + be0_ref[...]
    h = jnp.dot(h, w1_ref[...], preferred_element_type=jnp.float32)
    h = jnp.maximum(h + b1_ref[...], 0.0)
    h = (h - m1_ref[...]) * lax.rsqrt(v1_ref[...] + EPS) * g1_ref[...] + be1_ref[...]
    out_ref[...] = jnp.dot(h, w2_ref[...], preferred_element_type=jnp.float32) + b2_ref[...]


def _tc_mlp(xc, xn, bg, bb, bm, bv, w0c, w0n, b0, g0, be0, m0, v0,
            w1, b1, g1, be1, m1, v1, w2, b2):
    TB = 512
    grid = (B // TB,)
    row = lambda i: (i, 0)
    rep = lambda i: (0, 0)
    full = lambda a: pl.BlockSpec(a.shape, rep)
    return pl.pallas_call(
        _mlp_body,
        grid=grid,
        in_specs=[
            pl.BlockSpec((TB, F * D), row),
            pl.BlockSpec((TB, NUM), row),
            full(bg), full(bb), full(bm), full(bv),
            full(w0c), full(w0n), full(b0), full(g0), full(be0), full(m0), full(v0),
            full(w1), full(b1), full(g1), full(be1), full(m1), full(v1),
            full(w2), full(b2),
        ],
        out_specs=pl.BlockSpec((TB, 1), row),
        out_shape=jax.ShapeDtypeStruct((B, 1), jnp.float32),
    )(xc, xn, bg, bb, bm, bv, w0c, w0n, b0, g0, be0, m0, v0,
      w1, b1, g1, be1, m1, v1, w2, b2)


def kernel(x_categorical, x_numerical, emb_tables, bn_num_gamma, bn_num_beta,
           bn_num_mean, bn_num_var, w0, b0, g0, be0, m0, v0,
           w1, b1, g1, be1, m1, v1, w2, b2):
    idx = x_categorical.astype(jnp.int32)
    flat = idx + (jnp.arange(F, dtype=jnp.int32) * V)[None, :]
    flat = flat.reshape(NW, NCH, CH)
    table = emb_tables.reshape(F * V, D)
    rows = _sc_gather(table, flat)
    xc = rows.reshape(B, F * D)

    r2 = lambda a: a.reshape(1, -1)
    return _tc_mlp(
        xc, x_numerical,
        r2(bn_num_gamma), r2(bn_num_beta), r2(bn_num_mean), r2(bn_num_var),
        w0[:, :F * D].T, w0[:, F * D:].T, r2(b0), r2(g0), r2(be0), r2(m0), r2(v0),
        w1.T, r2(b1), r2(g1), r2(be1), r2(m1), r2(v1),
        w2.T, r2(b2),
    )
